# baseline (speedup 1.0000x reference)
import functools

import jax
import jax.numpy as jnp
from jax import lax
from jax.experimental import pallas as pl
from jax.experimental.pallas import tpu as pltpu
from jax.experimental.pallas import tpu_sc as plsc

_B, _S, _C = 4096, 26, 1000
_NW = 32
_BPW = _B // _NW
_IPW = _BPW * _S
_CW = 200
_NCW = _C // _CW
_NSTEP = _S * _NCW


def _scatter_val(buf, idx_v, step, vals):
    s = step // _NCW
    c0 = (step % _NCW) * _CW
    lane = lax.iota(jnp.int32, 16)
    svec = jnp.full((16,), s, jnp.int32)
    for g in range(_BPW // 16):
        blane = g * 16 + lane
        cols = plsc.load_gather(idx_v, [svec, blane])
        rel = cols - c0
        m = (rel >= 0) & (rel < _CW)
        plsc.store_scatter(buf, [rel, blane], vals, mask=m)


def _body(idx_hbm, out_hbm, idx_v, buf0, buf1, sem0, sem1):
    wid = lax.axis_index("s") * 2 + lax.axis_index("c")
    b0 = wid * _BPW
    pltpu.sync_copy(idx_hbm.at[:, pl.ds(b0, _BPW)], idx_v)

    bufs = (buf0, buf1)
    sems = (sem0, sem1)
    zeros16 = jnp.zeros((16,), jnp.float32)
    ones = jnp.full((16,), 1.0, jnp.float32)
    zeros = jnp.zeros((16,), jnp.float32)

    def make_zero_body(r):
        def zero_body(i, _):
            for c in range(_BPW // 16):
                bufs[r][i, pl.ds(c * 16, 16)] = zeros16
            return 0
        return zero_body

    def out_slice(step):
        s = step // _NCW
        c0 = (step % _NCW) * _CW
        return out_hbm.at[s, pl.ds(c0, _CW), pl.ds(b0, _BPW)]

    for b in range(2):
        lax.fori_loop(0, _CW, make_zero_body(b), 0)
        _scatter_val(bufs[b], idx_v, b, ones)
        pltpu.async_copy(bufs[b], out_slice(b), sems[b])

    def pair_body(p, _):
        for b in range(2):
            cur = p * 2 + b
            prev = cur - 2
            pltpu.make_async_copy(bufs[b], out_slice(prev), sems[b]).wait()
            _scatter_val(bufs[b], idx_v, prev, zeros)
            _scatter_val(bufs[b], idx_v, cur, ones)
            pltpu.async_copy(bufs[b], out_slice(cur), sems[b])
        return 0

    lax.fori_loop(1, _NSTEP // 2, pair_body, 0)

    for b in range(2):
        last = _NSTEP - 2 + b
        pltpu.make_async_copy(bufs[b], out_slice(last), sems[b]).wait()


@jax.jit
def _onehot(idx2d):
    mesh = plsc.VectorSubcoreMesh(core_axis_name="c", subcore_axis_name="s")
    res = pl.kernel(
        _body,
        out_type=jax.ShapeDtypeStruct((_S, _C, _B), jnp.float32),
        mesh=mesh,
        compiler_params=pltpu.CompilerParams(
            needs_layout_passes=False, skip_device_barrier=True),
        scratch_types=[
            pltpu.VMEM((_S, _BPW), jnp.int32),
            pltpu.VMEM((_CW, _BPW), jnp.float32),
            pltpu.VMEM((_CW, _BPW), jnp.float32),
            pltpu.SemaphoreType.DMA,
            pltpu.SemaphoreType.DMA,
        ],
    )(idx2d)
    return jnp.transpose(res, (2, 0, 1))


def kernel(inputs):
    return _onehot(inputs.astype(jnp.int32).T)
